# Initial kernel scaffold; baseline (speedup 1.0000x reference)
import jax
import jax.numpy as jnp
from jax.experimental import pallas as pl


def kernel(pos, neg, W_emb, W_ctx):
    raise NotImplementedError("write your pallas kernel here")


# speedup vs baseline: 3.8836x; 3.8836x over previous
import functools

import jax
import jax.numpy as jnp
from jax import lax
from jax.experimental import pallas as pl
from jax.experimental.pallas import tpu as pltpu
from jax.experimental.pallas import tpu_sc as plsc

B = 16384
NNEG = 20
V = 1000000
D = 64
L = 16

NW = 32
BPW = B // NW
C = 32
NCHUNK = BPW // C
GPC = C // L
NEGC = C * NNEG
IDXW = 80
NEGR = NEGC // IDXW
NEG_ROWS_TOT = B * NNEG // IDXW

LN2 = 0.6931471805599453
SQRTH = 0.7071067811865476


def _vlog(x):
    bits = lax.bitcast_convert_type(x, jnp.int32)
    e = lax.shift_right_logical(bits, 23) - 127
    m = lax.bitcast_convert_type(
        (bits & 0x007FFFFF) | 0x3F800000, jnp.float32)
    ef = e.astype(jnp.float32)
    small = m < (2.0 * SQRTH)
    ef = jnp.where(small, ef, ef + 1.0)
    m = jnp.where(small, m, 0.5 * m)
    r = m - 1.0
    z = r * r
    p = 7.0376836292e-2
    p = p * r + -1.1514610310e-1
    p = p * r + 1.1676998740e-1
    p = p * r + -1.2420140846e-1
    p = p * r + 1.4249322787e-1
    p = p * r + -1.6668057665e-1
    p = p * r + 2.0000714765e-1
    p = p * r + -2.4999993993e-1
    p = p * r + 3.3333331174e-1
    y = r * z * p - 0.5 * z + r
    return y + ef * LN2


def _make_sc_kernel():
    mesh = plsc.VectorSubcoreMesh(core_axis_name="c", subcore_axis_name="s")

    @functools.partial(
        pl.kernel,
        out_type=(
            jax.ShapeDtypeStruct((B,), jnp.float32),
            jax.ShapeDtypeStruct((B,), jnp.float32),
        ),
        mesh=mesh,
        compiler_params=pltpu.CompilerParams(
            use_tc_tiling_on_sc=False, needs_layout_passes=False),
        scratch_types=[
            pltpu.VMEM((C,), jnp.int32),
            pltpu.VMEM((C,), jnp.int32),
            pltpu.VMEM((C,), jnp.float32),
            pltpu.VMEM((NEGR, IDXW), jnp.int32),
            pltpu.VMEM((C, D), jnp.float32),
            pltpu.VMEM((C, D), jnp.float32),
            pltpu.VMEM((C, D), jnp.float32),
            pltpu.VMEM((NEGC, D), jnp.float32),
            pltpu.VMEM((C,), jnp.float32),
            pltpu.VMEM((C,), jnp.float32),
            pltpu.SemaphoreType.DMA,
        ],
    )
    def sc_kernel(pos_v_hbm, pos_u_hbm, w_hbm, neg_hbm, emb_hbm, ctx_hbm,
                  out1_hbm, out2_hbm,
                  idxv, idxu, wbuf, negidx, vrows, u1rows, u2rows, negrows,
                  o1, o2, sem):
        wid = lax.axis_index("s") * 2 + lax.axis_index("c")
        base = wid * BPW
        rbase = wid * (BPW * NNEG // IDXW)

        def chunk_body(ci, carry):
            b0 = pl.multiple_of(base + ci * C, C)
            r0 = rbase + ci * NEGR
            pltpu.sync_copy(pos_v_hbm.at[pl.ds(b0, C)], idxv)
            pltpu.sync_copy(pos_u_hbm.at[pl.ds(b0, C)], idxu)
            pltpu.sync_copy(w_hbm.at[pl.ds(b0, C)], wbuf)
            pltpu.sync_copy(neg_hbm.at[pl.ds(r0, NEGR)], negidx)
            cps = [
                pltpu.async_copy(emb_hbm.at[idxv], vrows, sem),
                pltpu.async_copy(emb_hbm.at[idxu], u1rows, sem),
                pltpu.async_copy(ctx_hbm.at[idxu], u2rows, sem),
            ]
            for k in range(NEGR):
                cps.append(pltpu.async_copy(
                    ctx_hbm.at[negidx.at[k]],
                    negrows.at[pl.ds(k * IDXW, IDXW)], sem))
            for cp in cps:
                cp.wait()

            zero = jnp.zeros((L,), jnp.float32)
            for g in range(GPC):
                bi = lax.iota(jnp.int32, L) + (g * L)
                nrow = bi * NNEG

                def dbody(dd, acc):
                    acc1, acc2, ts = acc
                    dv = jnp.full((L,), dd, jnp.int32)
                    v_d = plsc.load_gather(vrows, [bi, dv])
                    u1_d = plsc.load_gather(u1rows, [bi, dv])
                    u2_d = plsc.load_gather(u2rows, [bi, dv])
                    acc1 = acc1 + v_d * u1_d
                    acc2 = acc2 + v_d * u2_d
                    ts = tuple(
                        ts[n] + v_d * plsc.load_gather(negrows, [nrow + n, dv])
                        for n in range(NNEG))
                    return acc1, acc2, ts

                s1, s2, ts = lax.fori_loop(
                    0, D, dbody, (zero, zero, (zero,) * NNEG))

                smax = zero
                prod = jnp.full((L,), 1.0, jnp.float32)
                for t in ts:
                    smax = smax + jnp.maximum(t, 0.0)
                    prod = prod * (1.0 + jnp.exp(-jnp.abs(t)))
                f1 = 1.0 + jnp.exp(-jnp.abs(s1))
                f2 = 1.0 + jnp.exp(-jnp.abs(s2))
                wg = wbuf[pl.ds(g * L, L)]
                sc1 = (jnp.maximum(-s1, 0.0) + smax + _vlog(prod * f1)) * wg
                sc2 = (jnp.maximum(-s2, 0.0) + smax + _vlog(prod * f2)) * wg
                o1[pl.ds(g * L, L)] = sc1
                o2[pl.ds(g * L, L)] = sc2

            pltpu.sync_copy(o1, out1_hbm.at[pl.ds(b0, C)])
            pltpu.sync_copy(o2, out2_hbm.at[pl.ds(b0, C)])
            return carry

        lax.fori_loop(0, NCHUNK, chunk_body, 0)

    return sc_kernel


_SC_KERNEL = _make_sc_kernel()


@jax.jit
def kernel(pos, neg, W_emb, W_ctx):
    pos_v = pos[:, 0].astype(jnp.int32)
    pos_u = pos[:, 1].astype(jnp.int32)
    w = pos[:, 2]
    neg2d = neg.astype(jnp.int32).reshape(NEG_ROWS_TOT, IDXW)
    return _SC_KERNEL(pos_v, pos_u, w, neg2d, W_emb, W_ctx)
